# baseline (speedup 1.0000x reference)
import functools

import jax
import jax.numpy as jnp
from jax import lax
from jax.experimental import pallas as pl
from jax.experimental.pallas import tpu as pltpu
from jax.experimental.pallas import tpu_sc as plsc

VOCAB = 49408
HIDDEN = 768
MAX_POS = 77
BATCH = 1024
SEQ = 77

NC = 2
NS = 16
NW = NC * NS

B = BATCH * SEQ
RPW = B // NW
CHUNK = 32
NCHUNKS = RPW // CHUNK
LANES = 16
NVEC = HIDDEN // LANES


def _body(table_hbm, idx_hbm, pos_hbm, out_hbm, idx_v, pos_v, buf, sem):
    wid = lax.axis_index("s") * NC + lax.axis_index("c")
    row0 = wid * RPW

    pltpu.sync_copy(idx_hbm.at[wid], idx_v)
    pltpu.sync_copy(pos_hbm, pos_v)

    def chunk_body(c, _):
        pltpu.async_copy(table_hbm.at[idx_v.at[c]], buf, sem).wait()

        p0 = lax.rem(c * CHUNK, SEQ)

        def row_body(i, _):
            p = p0 + i
            p = jnp.where(p >= SEQ, p - SEQ, p)

            def col_body(j, _):
                sl = pl.ds(j * LANES, LANES)
                buf[i, sl] += pos_v[p, sl]
                return 0

            lax.fori_loop(0, NVEC, col_body, 0)
            return 0

        lax.fori_loop(0, CHUNK, row_body, 0)

        pltpu.sync_copy(buf, out_hbm.at[pl.ds(row0 + c * CHUNK, CHUNK)])
        return 0

    lax.fori_loop(0, NCHUNKS, chunk_body, 0)


_sc_call = functools.partial(
    pl.kernel,
    out_type=jax.ShapeDtypeStruct((B, HIDDEN), jnp.float32),
    mesh=plsc.VectorSubcoreMesh(
        core_axis_name="c", subcore_axis_name="s", num_cores=NC, num_subcores=NS
    ),
    scratch_types=[
        pltpu.VMEM((NCHUNKS, CHUNK), jnp.int32),
        pltpu.VMEM((MAX_POS, HIDDEN), jnp.float32),
        pltpu.VMEM((CHUNK, HIDDEN), jnp.float32),
        pltpu.SemaphoreType.DMA,
    ],
)(_body)


@jax.jit
def kernel(input_ids, token_table, pos_table):
    ids = input_ids.astype(jnp.int32).reshape(NW, NCHUNKS, CHUNK)
    out = _sc_call(token_table, ids, pos_table)
    return out.reshape(BATCH, SEQ, HIDDEN)


# speedup vs baseline: 2.0125x; 2.0125x over previous
import functools

import jax
import jax.numpy as jnp
from jax import lax
from jax.experimental import pallas as pl
from jax.experimental.pallas import tpu as pltpu
from jax.experimental.pallas import tpu_sc as plsc

VOCAB = 49408
HIDDEN = 768
MAX_POS = 77
BATCH = 1024
SEQ = 77

NC = 2
NS = 16
NW = NC * NS

B = BATCH * SEQ
BPW = BATCH // NW
LANES = 16
NVEC = HIDDEN // LANES
NBUF = 2


def _body(table_hbm, idx_hbm, oidx_hbm, pos_hbm, out_hbm,
          idx_v, oidx_v, pos_v, buf, gsem, ssem):
    wid = lax.axis_index("s") * NC + lax.axis_index("c")

    pltpu.sync_copy(idx_hbm.at[wid], idx_v)
    pltpu.sync_copy(oidx_hbm.at[wid], oidx_v)
    pltpu.sync_copy(pos_hbm, pos_v)

    def gather_start(p):
        m = lax.rem(p, NBUF)
        pltpu.async_copy(table_hbm.at[idx_v.at[p]], buf.at[m], gsem)

    def gather_wait(p):
        m = lax.rem(p, NBUF)
        pltpu.make_async_copy(table_hbm.at[idx_v.at[p]], buf.at[m], gsem).wait()

    def scatter_start(p):
        m = lax.rem(p, NBUF)
        pltpu.async_copy(buf.at[m], out_hbm.at[oidx_v.at[p]], ssem)

    def scatter_wait(p):
        m = lax.rem(p, NBUF)
        pltpu.make_async_copy(buf.at[m], out_hbm.at[oidx_v.at[p]], ssem).wait()

    gather_start(0)

    def chunk_body(p, _):
        @pl.when(p >= 1)
        def _():
            scatter_wait(p - 1)

        @pl.when(p + 1 < SEQ)
        def _():
            gather_start(p + 1)

        gather_wait(p)
        m = lax.rem(p, NBUF)

        def col_body(j, _):
            sl = pl.ds(j * LANES, LANES)
            pv = pos_v[p, sl]
            for b in range(BPW):
                buf[m, b, sl] += pv
            return 0

        lax.fori_loop(0, NVEC, col_body, 0)

        scatter_start(p)
        return 0

    lax.fori_loop(0, SEQ, chunk_body, 0)
    scatter_wait(SEQ - 1)


_sc_call = functools.partial(
    pl.kernel,
    out_type=jax.ShapeDtypeStruct((B, HIDDEN), jnp.float32),
    mesh=plsc.VectorSubcoreMesh(
        core_axis_name="c", subcore_axis_name="s", num_cores=NC, num_subcores=NS
    ),
    scratch_types=[
        pltpu.VMEM((SEQ, BPW), jnp.int32),
        pltpu.VMEM((SEQ, BPW), jnp.int32),
        pltpu.VMEM((MAX_POS, HIDDEN), jnp.float32),
        pltpu.VMEM((NBUF, BPW, HIDDEN), jnp.float32),
        pltpu.SemaphoreType.DMA,
        pltpu.SemaphoreType.DMA,
    ],
)(_body)


@jax.jit
def kernel(input_ids, token_table, pos_table):
    ids = input_ids.astype(jnp.int32).reshape(NW, BPW, SEQ).transpose(0, 2, 1)
    w = jnp.arange(NW, dtype=jnp.int32)[:, None, None]
    p = jnp.arange(SEQ, dtype=jnp.int32)[None, :, None]
    j = jnp.arange(BPW, dtype=jnp.int32)[None, None, :]
    oidx = (w * BPW + j) * SEQ + p
    out = _sc_call(token_table, ids, oidx, pos_table)
    return out.reshape(BATCH, SEQ, HIDDEN)
